# baseline (speedup 1.0000x reference)
import functools

import jax
import jax.numpy as jnp
from jax import lax
from jax.experimental import pallas as pl
from jax.experimental.pallas import tpu as pltpu
from jax.experimental.pallas import tpu_sc as plsc

_D = 32
_K = 8192
_NT = 512
_KT = 2048


def _argmin_body(x_ref, cbt2_ref, xsq_ref, csq_ref, idx_ref):
    x = x_ref[...]
    xsq = xsq_ref[...]

    def step(t, carry):
        best, besti = carry
        cbt2 = cbt2_ref[:, pl.ds(t * _KT, _KT)]
        mm2 = jnp.dot(x, cbt2, preferred_element_type=jnp.float32)
        d = (xsq + mm2) + csq_ref[:, pl.ds(t * _KT, _KT)]
        tm = jnp.min(d, axis=1, keepdims=True)
        iota = lax.broadcasted_iota(jnp.int32, d.shape, 1) + t * _KT
        ti = jnp.min(jnp.where(d == tm, iota, jnp.int32(_K)),
                     axis=1, keepdims=True)
        upd = tm < best
        return jnp.where(upd, tm, best), jnp.where(upd, ti, besti)

    best0 = jnp.full((_NT, 1), jnp.inf, jnp.float32)
    besti0 = jnp.zeros((_NT, 1), jnp.int32)
    _, besti = lax.fori_loop(0, _K // _KT, step, (best0, besti0))
    idx_ref[...] = besti


def _argmin_call(x, cbt2, xsq, csq):
    n = x.shape[0]
    grid = (n // _NT,)
    return pl.pallas_call(
        _argmin_body,
        grid=grid,
        in_specs=[
            pl.BlockSpec((_NT, _D), lambda i: (i, 0)),
            pl.BlockSpec((_D, _K), lambda i: (0, 0)),
            pl.BlockSpec((_NT, 1), lambda i: (i, 0)),
            pl.BlockSpec((1, _K), lambda i: (0, 0)),
        ],
        out_specs=pl.BlockSpec((_NT, 1), lambda i: (i, 0)),
        out_shape=jax.ShapeDtypeStruct((n, 1), jnp.int32),
    )(x, cbt2, xsq, csq)


def _make_gather(n):
    info = plsc.get_sparse_core_info()
    nw = info.num_cores * info.num_subcores
    b_per_w = n // nw
    mesh = plsc.VectorSubcoreMesh(core_axis_name="c", subcore_axis_name="s")

    @functools.partial(
        pl.kernel,
        mesh=mesh,
        out_type=jax.ShapeDtypeStruct((n, _D), jnp.float32),
        compiler_params=pltpu.CompilerParams(use_tc_tiling_on_sc=False),
        scratch_types=[
            pltpu.VMEM((b_per_w,), jnp.int32),
            pltpu.VMEM((b_per_w, _D), jnp.float32),
            pltpu.SemaphoreType.DMA,
        ],
    )
    def gather(table_hbm, idx_hbm, out_hbm, idx_v, rows_v, sem):
        wid = lax.axis_index("s") * info.num_cores + lax.axis_index("c")
        base = wid * b_per_w
        pltpu.sync_copy(idx_hbm.at[pl.ds(base, b_per_w)], idx_v)
        pltpu.async_copy(table_hbm.at[idx_v], rows_v, sem).wait()
        pltpu.sync_copy(rows_v, out_hbm.at[pl.ds(base, b_per_w)])

    return gather


def kernel(inputs, codebook):
    B, C, H, W = inputs.shape
    n = B * H * W
    x = jnp.transpose(inputs, (0, 2, 3, 1)).reshape(-1, C)
    xsq = jnp.sum(x ** 2, axis=1, keepdims=True)
    csq = jnp.sum(codebook ** 2, axis=1)[None, :]
    cbt2 = -2.0 * codebook.T

    idx = _argmin_call(x, cbt2, xsq, csq).reshape(-1)
    rows = jnp.take(codebook, idx, axis=0)

    quantized = rows.reshape(B, H, W, C).transpose(0, 3, 1, 2)
    return inputs + (quantized - inputs)


# speedup vs baseline: 1.2304x; 1.2304x over previous
import functools

import jax
import jax.numpy as jnp
from jax import lax
from jax.experimental import pallas as pl
from jax.experimental.pallas import tpu as pltpu
from jax.experimental.pallas import tpu_sc as plsc

_D = 32
_K = 8192
_NT = 512
_KT = 2048


def _argmin_body(x_ref, cbaug_ref, idx_ref):
    x = x_ref[...]

    def step(t, carry):
        best, besti = carry
        cbaug = cbaug_ref[:, pl.ds(t * _KT, _KT)]
        d = jnp.dot(x, cbaug, preferred_element_type=jnp.float32)
        tm = jnp.min(d, axis=1, keepdims=True)
        iota = lax.broadcasted_iota(jnp.int32, d.shape, 1) + t * _KT
        ti = jnp.min(jnp.where(d == tm, iota, jnp.int32(_K)),
                     axis=1, keepdims=True)
        upd = tm < best
        return jnp.where(upd, tm, best), jnp.where(upd, ti, besti)

    best0 = jnp.full((_NT, 1), jnp.inf, jnp.float32)
    besti0 = jnp.zeros((_NT, 1), jnp.int32)
    _, besti = lax.fori_loop(0, _K // _KT, step, (best0, besti0))
    idx_ref[...] = besti


def _argmin_call(x_aug, cb_aug):
    n = x_aug.shape[0]
    grid = (n // _NT,)
    return pl.pallas_call(
        _argmin_body,
        grid=grid,
        in_specs=[
            pl.BlockSpec((_NT, _D + 2), lambda i: (i, 0)),
            pl.BlockSpec((_D + 2, _K), lambda i: (0, 0)),
        ],
        out_specs=pl.BlockSpec((_NT, 1), lambda i: (i, 0)),
        out_shape=jax.ShapeDtypeStruct((n, 1), jnp.int32),
    )(x_aug, cb_aug)


def _make_gather(n):
    info = plsc.get_sparse_core_info()
    nw = info.num_cores * info.num_subcores
    b_per_w = n // nw
    mesh = plsc.VectorSubcoreMesh(core_axis_name="c", subcore_axis_name="s")

    @functools.partial(
        pl.kernel,
        mesh=mesh,
        out_type=jax.ShapeDtypeStruct((n, _D), jnp.float32),
        compiler_params=pltpu.CompilerParams(use_tc_tiling_on_sc=False),
        scratch_types=[
            pltpu.VMEM((b_per_w,), jnp.int32),
            pltpu.VMEM((b_per_w, _D), jnp.float32),
            pltpu.SemaphoreType.DMA,
        ],
    )
    def gather(table_hbm, idx_hbm, out_hbm, idx_v, rows_v, sem):
        wid = lax.axis_index("s") * info.num_cores + lax.axis_index("c")
        base = wid * b_per_w
        pltpu.sync_copy(idx_hbm.at[pl.ds(base, b_per_w)], idx_v)
        pltpu.async_copy(table_hbm.at[idx_v], rows_v, sem).wait()
        pltpu.sync_copy(rows_v, out_hbm.at[pl.ds(base, b_per_w)])

    return gather


def kernel(inputs, codebook):
    B, C, H, W = inputs.shape
    n = B * H * W
    x = jnp.transpose(inputs, (0, 2, 3, 1)).reshape(-1, C)
    csq = jnp.sum(codebook ** 2, axis=1)[None, :]
    csq_hi = csq.astype(jnp.bfloat16).astype(jnp.float32)
    csq_lo = csq - csq_hi
    x_aug = jnp.concatenate([x, jnp.ones((n, 2), jnp.float32)], axis=1)
    cb_aug = jnp.concatenate([-2.0 * codebook.T, csq_hi, csq_lo], axis=0)

    idx = _argmin_call(x_aug, cb_aug).reshape(-1)
    rows = _make_gather(n)(codebook, idx)

    quantized = rows.reshape(B, H, W, C).transpose(0, 3, 1, 2)
    return inputs + (quantized - inputs)
